# Initial kernel scaffold; baseline (speedup 1.0000x reference)
import jax
import jax.numpy as jnp
from jax.experimental import pallas as pl


def kernel(ppg, group_labels, pca_matrix, pca_mean, W_pca, W_fft, W_wav, keys, prompts, gumbel):
    raise NotImplementedError("write your pallas kernel here")


# speedup vs baseline: 14.1089x; 14.1089x over previous
import functools
import math

import jax
import jax.numpy as jnp
from jax import lax
from jax.experimental import pallas as pl
from jax.experimental.pallas import tpu as pltpu

_B = 1024
_DATA = 4096
_EMB = 768
_PCA = 256
_NPOOL = 64
_NCOMP = _DATA // 2 + 1
_JPAD = 2176
_TTILE = 512
_BTILE = 256
_SQRT_HALF = 1.0 / math.sqrt(2.0)
_INV_SQRT_DK = 1.0 / math.sqrt(float(_EMB))


def _pre_small_body(k0_ref, k2_ref, wp_ref, pmat_ref, pmean_ref, wwav_ref,
                    pr_ref, kp_out, kw_out, bias_out, psum_out):
    f32 = jnp.float32
    kp = jnp.dot(k0_ref[...], wp_ref[...], preferred_element_type=f32)
    kpt = lax.dot_general(pmat_ref[...], kp, (((0,), (1,)), ((), ())),
                          preferred_element_type=f32)
    kp_out[...] = kpt
    bias_out[...] = -jnp.dot(pmean_ref[...], kpt, preferred_element_type=f32)
    krt = lax.dot_general(wwav_ref[...], k2_ref[...], (((0,), (1,)), ((), ())),
                          preferred_element_type=f32)
    s = jnp.float32(_SQRT_HALF)
    a = krt[0:128, :]
    off, m = 128, 128
    for _ in range(5):
        d = krt[off:off + m, :]
        even = (a - d) * s
        odd = (a + d) * s
        a = jnp.concatenate([even[:, None, :], odd[:, None, :]],
                            axis=1).reshape(2 * m, _NPOOL)
        off += m
        m *= 2
    kw_out[...] = a
    psum_out[...] = jnp.sum(pr_ref[...], axis=0, keepdims=True)


def _pre_fft_body(k1_ref, wf_ref, kf_out, kf_scr):
    f32 = jnp.float32
    step = pl.program_id(0)

    @pl.when(step == 0)
    def _():
        kf_scr[...] = jnp.dot(k1_ref[...], wf_ref[...],
                              preferred_element_type=f32)

    t0 = step * _TTILE
    nj1 = _JPAD // 64
    j1 = lax.broadcasted_iota(jnp.int32, (nj1, _TTILE), 0)
    tg = t0 + lax.broadcasted_iota(jnp.int32, (nj1, _TTILE), 1)
    ang_a = ((j1 * tg) & 63).astype(f32) * jnp.float32(2.0 * math.pi / 64.0)
    cos_a, sin_a = jnp.cos(ang_a), jnp.sin(ang_a)
    j2 = lax.broadcasted_iota(jnp.int32, (64, _TTILE), 0)
    tg2 = t0 + lax.broadcasted_iota(jnp.int32, (64, _TTILE), 1)
    ang_b = ((j2 * tg2) & (_DATA - 1)).astype(f32) * jnp.float32(
        2.0 * math.pi / _DATA)
    cos_b, sin_b = jnp.cos(ang_b), jnp.sin(ang_b)
    ctile = (cos_a[:, None, :] * cos_b[None, :, :]
             - sin_a[:, None, :] * sin_b[None, :, :]).reshape(_JPAD, _TTILE)
    kf_out[...] = lax.dot_general(ctile, kf_scr[...], (((0,), (1,)), ((), ())),
                                  preferred_element_type=f32)


def _branch_score(z_cos, g):
    z = z_cos + g
    zmax = jnp.max(z, axis=1, keepdims=True)
    return jnp.max(jnp.where(z >= zmax, z_cos, -jnp.inf), axis=1,
                   keepdims=True)


def _main_body(flat_ref, kp_ref, kf_ref, kw_ref, bias_ref, psum_ref,
               g0_ref, g1_ref, g2_ref, out_ref, sim_ref, ent_ref, acc_ref):
    f32 = jnp.float32
    step = pl.program_id(0)
    nsteps = pl.num_programs(0)
    x = flat_ref[...]
    inv = jnp.float32(_INV_SQRT_DK)
    cos0 = (jnp.dot(x, kp_ref[...], preferred_element_type=f32)
            + bias_ref[...]) * inv
    cos1 = jnp.dot(x, kf_ref[...], preferred_element_type=f32) * inv
    cos2 = jnp.dot(x, kw_ref[...], preferred_element_type=f32) * inv
    ms0 = _branch_score(cos0, g0_ref[...])
    ms1 = _branch_score(cos1, g1_ref[...])
    ms2 = _branch_score(cos2, g2_ref[...])
    mm = jnp.maximum(jnp.maximum(ms0, ms1), ms2)
    e0 = jnp.exp(ms0 - mm)
    e1 = jnp.exp(ms1 - mm)
    e2 = jnp.exp(ms2 - mm)
    ssum = e0 + e1 + e2
    w0, w1, w2 = e0 / ssum, e1 / ssum, e2 / ssum
    eps = jnp.float32(1e-10)
    ent_rows = -(w0 * jnp.log(w0 + eps) + w1 * jnp.log(w1 + eps)
                 + w2 * jnp.log(w2 + eps))
    part_ms = jnp.sum(ms0 + ms1 + ms2)
    part_ent = jnp.sum(ent_rows)

    @pl.when(step == 0)
    def _():
        acc_ref[0] = 0.0
        acc_ref[1] = 0.0

    acc_ref[0] = acc_ref[0] + part_ms
    acc_ref[1] = acc_ref[1] + part_ent
    out_ref[...] = x + psum_ref[...]

    @pl.when(step == nsteps - 1)
    def _():
        sim_ref[0, 0] = jnp.maximum(1.0 - acc_ref[0] / (3.0 * _B), 0.0)
        ent_ref[0, 0] = -(acc_ref[1] / _B)


@functools.partial(jax.jit, static_argnums=())
def _run(flat, keys0, keys1, keys2, pca_matrix, pmean_row, W_pca, Wf_pad,
         W_wav, prompts2d, g0, g1, g2):
    f32 = jnp.float32
    kpt, kwt, bias, psum = pl.pallas_call(
        _pre_small_body,
        out_shape=(
            jax.ShapeDtypeStruct((_DATA, _NPOOL), f32),
            jax.ShapeDtypeStruct((_DATA, _NPOOL), f32),
            jax.ShapeDtypeStruct((1, _NPOOL), f32),
            jax.ShapeDtypeStruct((1, _DATA), f32),
        ),
    )(keys0, keys2, W_pca, pca_matrix, pmean_row, W_wav, prompts2d)

    nfstep = _DATA // _TTILE
    kft = pl.pallas_call(
        _pre_fft_body,
        grid=(nfstep,),
        in_specs=[
            pl.BlockSpec((_NPOOL, _EMB), lambda s: (0, 0)),
            pl.BlockSpec((_EMB, _JPAD), lambda s: (0, 0)),
        ],
        out_specs=pl.BlockSpec((_TTILE, _NPOOL), lambda s: (s, 0)),
        out_shape=jax.ShapeDtypeStruct((_DATA, _NPOOL), f32),
        scratch_shapes=[pltpu.VMEM((_NPOOL, _JPAD), f32)],
    )(keys1, Wf_pad)

    nbstep = _B // _BTILE
    prompted, sim, ent = pl.pallas_call(
        _main_body,
        grid=(nbstep,),
        in_specs=[
            pl.BlockSpec((_BTILE, _DATA), lambda s: (s, 0)),
            pl.BlockSpec((_DATA, _NPOOL), lambda s: (0, 0)),
            pl.BlockSpec((_DATA, _NPOOL), lambda s: (0, 0)),
            pl.BlockSpec((_DATA, _NPOOL), lambda s: (0, 0)),
            pl.BlockSpec((1, _NPOOL), lambda s: (0, 0)),
            pl.BlockSpec((1, _DATA), lambda s: (0, 0)),
            pl.BlockSpec((_BTILE, _NPOOL), lambda s: (s, 0)),
            pl.BlockSpec((_BTILE, _NPOOL), lambda s: (s, 0)),
            pl.BlockSpec((_BTILE, _NPOOL), lambda s: (s, 0)),
        ],
        out_specs=(
            pl.BlockSpec((_BTILE, _DATA), lambda s: (s, 0)),
            pl.BlockSpec(memory_space=pltpu.SMEM),
            pl.BlockSpec(memory_space=pltpu.SMEM),
        ),
        out_shape=(
            jax.ShapeDtypeStruct((_B, _DATA), f32),
            jax.ShapeDtypeStruct((1, 1), f32),
            jax.ShapeDtypeStruct((1, 1), f32),
        ),
        scratch_shapes=[pltpu.SMEM((2,), f32)],
    )(flat, kpt, kft, kwt, bias, psum, g0, g1, g2)
    return prompted, sim, ent


def kernel(ppg, group_labels, pca_matrix, pca_mean, W_pca, W_fft, W_wav,
           keys, prompts, gumbel):
    del group_labels
    flat = ppg.reshape(_B, _DATA)
    keys0 = keys[:, 0, :]
    keys1 = keys[:, 1, :]
    keys2 = keys[:, 2, :]
    Wf_pad = jnp.concatenate(
        [W_fft, jnp.zeros((_EMB, _JPAD - _NCOMP), W_fft.dtype)], axis=1)
    prompts2d = prompts[:, 0, :]
    pmean_row = pca_mean.reshape(1, _DATA)
    g0 = gumbel[:, 0, :]
    g1 = gumbel[:, 1, :]
    g2 = gumbel[:, 2, :]
    prompted, sim, ent = _run(flat, keys0, keys1, keys2, pca_matrix,
                              pmean_row, W_pca, Wf_pad, W_wav, prompts2d,
                              g0, g1, g2)
    return (prompted.reshape(_B, 1, _DATA), sim.reshape(()), ent.reshape(()))


# speedup vs baseline: 20.1059x; 1.4251x over previous
import functools
import math

import jax
import jax.numpy as jnp
from jax import lax
from jax.experimental import pallas as pl
from jax.experimental.pallas import tpu as pltpu

_B = 1024
_DATA = 4096
_EMB = 768
_PCA = 256
_NPOOL = 64
_NCOMP = _DATA // 2 + 1
_JMAIN = 2048
_TTILE = 512
_BTILE = 64
_SQRT_HALF = 1.0 / math.sqrt(2.0)
_INV_SQRT_DK = 1.0 / math.sqrt(float(_EMB))


def _pre_small_body(keys_ref, wp_ref, pmat_ref, pmean_ref, wwav_ref,
                    pr_ref, kp_out, kw_out, bias_out, psum_out):
    f32 = jnp.float32
    k0 = keys_ref[:, 0, :]
    k2 = keys_ref[:, 2, :]
    kp = jnp.dot(k0, wp_ref[...], preferred_element_type=f32)
    kpt = lax.dot_general(pmat_ref[...], kp, (((0,), (1,)), ((), ())),
                          preferred_element_type=f32)
    kp_out[...] = kpt
    bias_out[...] = -jnp.dot(pmean_ref[...], kpt, preferred_element_type=f32)
    krt = lax.dot_general(wwav_ref[...], k2, (((0,), (1,)), ((), ())),
                          preferred_element_type=f32)
    s = jnp.float32(_SQRT_HALF)
    a = krt[0:128, :]
    off, m = 128, 128
    for _ in range(5):
        d = krt[off:off + m, :]
        even = (a - d) * s
        odd = (a + d) * s
        a = jnp.concatenate([even[:, None, :], odd[:, None, :]],
                            axis=1).reshape(2 * m, _NPOOL)
        off += m
        m *= 2
    kw_out[...] = a
    psum_out[...] = jnp.sum(pr_ref[:, 0, :], axis=0, keepdims=True)


def _pre_fft_body(keys_ref, wf_ref, kf_out, kf_scr):
    f32 = jnp.float32
    step = pl.program_id(0)

    @pl.when(step == 0)
    def _():
        kf_scr[...] = jnp.dot(keys_ref[:, 1, :], wf_ref[...],
                              preferred_element_type=f32)

    t0 = step * _TTILE
    nj1 = _JMAIN // 64
    j1 = lax.broadcasted_iota(jnp.int32, (nj1, _TTILE), 0)
    tg = t0 + lax.broadcasted_iota(jnp.int32, (nj1, _TTILE), 1)
    ang_a = ((j1 * tg) & 63).astype(f32) * jnp.float32(2.0 * math.pi / 64.0)
    cos_a, sin_a = jnp.cos(ang_a), jnp.sin(ang_a)
    j2 = lax.broadcasted_iota(jnp.int32, (64, _TTILE), 0)
    tg2 = t0 + lax.broadcasted_iota(jnp.int32, (64, _TTILE), 1)
    ang_b = ((j2 * tg2) & (_DATA - 1)).astype(f32) * jnp.float32(
        2.0 * math.pi / _DATA)
    cos_b, sin_b = jnp.cos(ang_b), jnp.sin(ang_b)
    ctile = (cos_a[:, None, :] * cos_b[None, :, :]
             - sin_a[:, None, :] * sin_b[None, :, :]).reshape(_JMAIN, _TTILE)
    kf_main = kf_scr[:, :_JMAIN]
    kf_ny = kf_scr[:, _JMAIN:_NCOMP]
    acc = lax.dot_general(ctile, kf_main, (((0,), (1,)), ((), ())),
                          preferred_element_type=f32)
    trow = lax.broadcasted_iota(jnp.int32, (_TTILE, 1), 0) + t0
    sign = (1 - 2 * (trow & 1)).astype(f32)
    acc = acc + lax.dot_general(sign, kf_ny, (((1,), (1,)), ((), ())),
                                preferred_element_type=f32)
    kf_out[...] = acc


def _branch_score(z_cos, g):
    z = z_cos + g
    zmax = jnp.max(z, axis=1, keepdims=True)
    return jnp.max(jnp.where(z >= zmax, z_cos, -jnp.inf), axis=1,
                   keepdims=True)


def _main_body(ppg_ref, kp_ref, kf_ref, kw_ref, bias_ref, psum_ref,
               g_ref, out_ref, sim_ref, ent_ref, acc_ref):
    f32 = jnp.float32
    step = pl.program_id(0)
    nsteps = pl.num_programs(0)
    x = ppg_ref[:, 0, :]
    inv = jnp.float32(_INV_SQRT_DK)
    cos0 = (jnp.dot(x, kp_ref[...], preferred_element_type=f32)
            + bias_ref[...]) * inv
    cos1 = jnp.dot(x, kf_ref[...], preferred_element_type=f32) * inv
    cos2 = jnp.dot(x, kw_ref[...], preferred_element_type=f32) * inv
    ms0 = _branch_score(cos0, g_ref[:, 0, :])
    ms1 = _branch_score(cos1, g_ref[:, 1, :])
    ms2 = _branch_score(cos2, g_ref[:, 2, :])
    mm = jnp.maximum(jnp.maximum(ms0, ms1), ms2)
    e0 = jnp.exp(ms0 - mm)
    e1 = jnp.exp(ms1 - mm)
    e2 = jnp.exp(ms2 - mm)
    ssum = e0 + e1 + e2
    w0, w1, w2 = e0 / ssum, e1 / ssum, e2 / ssum
    eps = jnp.float32(1e-10)
    ent_rows = -(w0 * jnp.log(w0 + eps) + w1 * jnp.log(w1 + eps)
                 + w2 * jnp.log(w2 + eps))
    part_ms = jnp.sum(ms0 + ms1 + ms2)
    part_ent = jnp.sum(ent_rows)

    @pl.when(step == 0)
    def _():
        acc_ref[0] = 0.0
        acc_ref[1] = 0.0

    acc_ref[0] = acc_ref[0] + part_ms
    acc_ref[1] = acc_ref[1] + part_ent
    out_ref[:, 0, :] = x + psum_ref[...]

    @pl.when(step == nsteps - 1)
    def _():
        sim_ref[0, 0] = jnp.maximum(1.0 - acc_ref[0] / (3.0 * _B), 0.0)
        ent_ref[0, 0] = -(acc_ref[1] / _B)


@jax.jit
def _run(ppg, keys, pca_matrix, pmean_row, W_pca, W_fft, W_wav, prompts,
         gumbel):
    f32 = jnp.float32
    kpt, kwt, bias, psum = pl.pallas_call(
        _pre_small_body,
        out_shape=(
            jax.ShapeDtypeStruct((_DATA, _NPOOL), f32),
            jax.ShapeDtypeStruct((_DATA, _NPOOL), f32),
            jax.ShapeDtypeStruct((1, _NPOOL), f32),
            jax.ShapeDtypeStruct((1, _DATA), f32),
        ),
    )(keys, W_pca, pca_matrix, pmean_row, W_wav, prompts)

    nfstep = _DATA // _TTILE
    kft = pl.pallas_call(
        _pre_fft_body,
        grid=(nfstep,),
        in_specs=[
            pl.BlockSpec((_NPOOL, 3, _EMB), lambda s: (0, 0, 0)),
            pl.BlockSpec((_EMB, _NCOMP), lambda s: (0, 0)),
        ],
        out_specs=pl.BlockSpec((_TTILE, _NPOOL), lambda s: (s, 0)),
        out_shape=jax.ShapeDtypeStruct((_DATA, _NPOOL), f32),
        scratch_shapes=[pltpu.VMEM((_NPOOL, _NCOMP), f32)],
    )(keys, W_fft)

    nbstep = _B // _BTILE
    prompted, sim, ent = pl.pallas_call(
        _main_body,
        grid=(nbstep,),
        in_specs=[
            pl.BlockSpec((_BTILE, 1, _DATA), lambda s: (s, 0, 0)),
            pl.BlockSpec((_DATA, _NPOOL), lambda s: (0, 0)),
            pl.BlockSpec((_DATA, _NPOOL), lambda s: (0, 0)),
            pl.BlockSpec((_DATA, _NPOOL), lambda s: (0, 0)),
            pl.BlockSpec((1, _NPOOL), lambda s: (0, 0)),
            pl.BlockSpec((1, _DATA), lambda s: (0, 0)),
            pl.BlockSpec((_BTILE, 3, _NPOOL), lambda s: (s, 0, 0)),
        ],
        out_specs=(
            pl.BlockSpec((_BTILE, 1, _DATA), lambda s: (s, 0, 0)),
            pl.BlockSpec(memory_space=pltpu.SMEM),
            pl.BlockSpec(memory_space=pltpu.SMEM),
        ),
        out_shape=(
            jax.ShapeDtypeStruct((_B, 1, _DATA), f32),
            jax.ShapeDtypeStruct((1, 1), f32),
            jax.ShapeDtypeStruct((1, 1), f32),
        ),
        scratch_shapes=[pltpu.SMEM((2,), f32)],
    )(ppg, kpt, kft, kwt, bias, psum, gumbel)
    return prompted, sim, ent


def kernel(ppg, group_labels, pca_matrix, pca_mean, W_pca, W_fft, W_wav,
           keys, prompts, gumbel):
    del group_labels
    pmean_row = pca_mean.reshape(1, _DATA)
    prompted, sim, ent = _run(ppg, keys, pca_matrix, pmean_row, W_pca,
                              W_fft, W_wav, prompts, gumbel)
    return (prompted, sim.reshape(()), ent.reshape(()))
